# baseline (speedup 1.0000x reference)
import functools

import jax
import jax.numpy as jnp
from jax import lax
from jax.experimental import pallas as pl
from jax.experimental.pallas import tpu as pltpu
from jax.experimental.pallas import tpu_sc as plsc

NR_CATE = 26
BATCH = 16384
NR_FEAT = 128

NC = 2
NS = 16
L = 16
NW = NC * NS
BPW = BATCH // NW
CHUNK = 128
NCH = BPW // CHUNK


def _maskout_sc(x_flat, label):
    mesh = plsc.VectorSubcoreMesh(core_axis_name="c", subcore_axis_name="s")

    @functools.partial(
        pl.kernel,
        mesh=mesh,
        out_type=jax.ShapeDtypeStruct((BATCH, NR_FEAT), jnp.float32),
        scratch_types=[
            pltpu.VMEM((BPW,), jnp.int32),
            pltpu.VMEM((NCH, CHUNK), jnp.int32),
            pltpu.VMEM((BPW, NR_FEAT), jnp.float32),
            pltpu.SemaphoreType.DMA,
        ],
    )
    def k(x_hbm, label_hbm, out_hbm, label_v, idx_v, rows_v, sem):
        wid = lax.axis_index("s") * NC + lax.axis_index("c")
        base = wid * BPW
        pltpu.sync_copy(label_hbm.at[pl.ds(base, BPW)], label_v)
        lane = lax.iota(jnp.int32, L)
        for c in range(NCH):
            for j in range(CHUNK // L):
                off = c * CHUNK + j * L
                lab = label_v[pl.ds(off, L)]
                idx_v[c, pl.ds(j * L, L)] = (base + off + lane) * NR_CATE + lab
        copies = [
            pltpu.async_copy(
                x_hbm.at[idx_v.at[c]], rows_v.at[pl.ds(c * CHUNK, CHUNK)], sem
            )
            for c in range(NCH)
        ]
        for cp in copies:
            cp.wait()
        pltpu.sync_copy(rows_v, out_hbm.at[pl.ds(base, BPW)])

    return k(x_flat, label)


def kernel(x, label):
    x_flat = x.reshape(BATCH * NR_CATE, NR_FEAT)
    return _maskout_sc(x_flat, label)


# speedup vs baseline: 1.9453x; 1.9453x over previous
import functools

import jax
import jax.numpy as jnp
from jax import lax
from jax.experimental import pallas as pl
from jax.experimental.pallas import tpu as pltpu
from jax.experimental.pallas import tpu_sc as plsc

NR_CATE = 26
BATCH = 16384
NR_FEAT = 128

NC = 2
NS = 16
NW = NC * NS
BPW = BATCH // NW
UNROLL = 16


def kernel(x, label):
    mesh = plsc.VectorSubcoreMesh(core_axis_name="c", subcore_axis_name="s")

    @functools.partial(
        pl.kernel,
        mesh=mesh,
        out_type=jax.ShapeDtypeStruct((BATCH, NR_FEAT), jnp.float32),
        scratch_types=[
            pltpu.VMEM((BPW,), jnp.int32),
            pltpu.VMEM((BPW, NR_FEAT), jnp.float32),
            pltpu.SemaphoreType.DMA,
        ],
    )
    def k(x_hbm, label_hbm, out_hbm, label_v, rows_v, sem):
        wid = lax.axis_index("s") * NC + lax.axis_index("c")
        base = wid * BPW
        pltpu.sync_copy(label_hbm.at[pl.ds(base, BPW)], label_v)

        def body(i, carry):
            lab_vec = label_v[pl.ds(i * UNROLL, UNROLL)]
            for u in range(UNROLL):
                ii = i * UNROLL + u
                pltpu.async_copy(
                    x_hbm.at[base + ii, lab_vec[u]], rows_v.at[ii], sem
                )
            return carry

        lax.fori_loop(0, BPW // UNROLL, body, 0)
        pltpu.make_async_copy(
            out_hbm.at[pl.ds(base, BPW)], rows_v, sem
        ).wait()
        pltpu.sync_copy(rows_v, out_hbm.at[pl.ds(base, BPW)])

    return k(x, label)
